# baseline (speedup 1.0000x reference)
import jax
import jax.numpy as jnp
from jax.experimental import pallas as pl

_B, _C, _H, _W = 8, 128, 128, 128
_F = 32
_HW = _H * _W
_S = 4096


def _gemm_kernel(idx_ref, w_ref, x_ref, o_ref):
    idxv = idx_ref[0, :]
    cols = jax.lax.broadcasted_iota(jnp.int32, (_C, _C), 1)
    onehot_t = (idxv[:, None] == cols).astype(jnp.float32)
    w_eff = jnp.dot(w_ref[...], onehot_t, preferred_element_type=jnp.float32)
    w_hi = w_eff.astype(jnp.bfloat16)
    w_lo = (w_eff - w_hi.astype(jnp.float32)).astype(jnp.bfloat16)
    xb = x_ref[0]
    x_hi = xb.astype(jnp.bfloat16)
    x_lo = (xb - x_hi.astype(jnp.float32)).astype(jnp.bfloat16)
    acc = jnp.dot(w_hi, x_hi, preferred_element_type=jnp.float32)
    acc += jnp.dot(w_hi, x_lo, preferred_element_type=jnp.float32)
    acc += jnp.dot(w_lo, x_hi, preferred_element_type=jnp.float32)
    o_ref[0] = acc


def kernel(x, weights, channel_idx):
    xf = x.reshape(_B, _C, _HW)
    w2 = weights.reshape(_F, _C)
    idx2 = channel_idx.reshape(1, _C)
    out = pl.pallas_call(
        _gemm_kernel,
        grid=(_B, _HW // _S),
        in_specs=[
            pl.BlockSpec((1, _C), lambda b, s: (0, 0)),
            pl.BlockSpec((_F, _C), lambda b, s: (0, 0)),
            pl.BlockSpec((1, _C, _S), lambda b, s: (b, 0, s)),
        ],
        out_specs=pl.BlockSpec((1, _F, _S), lambda b, s: (b, 0, s)),
        out_shape=jax.ShapeDtypeStruct((_B, _F, _HW), jnp.float32),
    )(idx2, w2, xf)
    return out.reshape(_B, _F, _H, _W)


# speedup vs baseline: 1.0368x; 1.0368x over previous
import jax
import jax.numpy as jnp
from jax.experimental import pallas as pl

_B, _C, _H, _W = 8, 128, 128, 128
_F = 32
_HW = _H * _W
_S = 4096


def _gemm_kernel(idx_ref, w_ref, x_ref, o_ref):
    idxv = idx_ref[0, :]
    cols = jax.lax.broadcasted_iota(jnp.int32, (_C, _C), 1)
    onehot_t = (idxv[:, None] == cols).astype(jnp.float32)
    w_eff = jnp.dot(w_ref[...], onehot_t, preferred_element_type=jnp.float32)
    w_hi = w_eff.astype(jnp.bfloat16)
    x_hi = x_ref[0].astype(jnp.bfloat16)
    o_ref[0] = jnp.dot(w_hi, x_hi, preferred_element_type=jnp.float32)


def kernel(x, weights, channel_idx):
    xf = x.reshape(_B, _C, _HW)
    w2 = weights.reshape(_F, _C)
    idx2 = channel_idx.reshape(1, _C)
    out = pl.pallas_call(
        _gemm_kernel,
        grid=(_B, _HW // _S),
        in_specs=[
            pl.BlockSpec((1, _C), lambda b, s: (0, 0)),
            pl.BlockSpec((_F, _C), lambda b, s: (0, 0)),
            pl.BlockSpec((1, _C, _S), lambda b, s: (b, 0, s)),
        ],
        out_specs=pl.BlockSpec((1, _F, _S), lambda b, s: (b, 0, s)),
        out_shape=jax.ShapeDtypeStruct((_B, _F, _HW), jnp.float32),
    )(idx2, w2, xf)
    return out.reshape(_B, _F, _H, _W)


# speedup vs baseline: 4.6573x; 4.4921x over previous
import jax
import jax.numpy as jnp
from jax.experimental import pallas as pl

_B, _C, _H, _W = 8, 128, 128, 128
_F = 32
_P = 2


def _gemm_kernel(idx_ref, w_ref, x_ref, o_ref):
    idxv = idx_ref[0, :]
    cols = jax.lax.broadcasted_iota(jnp.int32, (_C, _C), 1)
    onehot_t = (idxv[:, None] == cols).astype(jnp.float32)
    w_eff = jnp.dot(w_ref[...], onehot_t, preferred_element_type=jnp.float32)
    w_hi = w_eff.astype(jnp.bfloat16)
    zero = jnp.zeros((_F, _C), jnp.bfloat16)
    w2 = jnp.concatenate(
        [jnp.concatenate([w_hi if i == j else zero for j in range(_P)], axis=1)
         for i in range(_P)], axis=0)

    xt = jnp.swapaxes(x_ref[0].astype(jnp.bfloat16), 0, 1)
    xr = xt.reshape(_H * _C, _W)
    outs = []
    for h2 in range(_H // _P):
        seg = xr[h2 * _P * _C:(h2 + 1) * _P * _C, :]
        outs.append(jnp.dot(w2, seg, preferred_element_type=jnp.float32))
    ot = jnp.concatenate(outs, axis=0).reshape(_H, _F, _W)
    o_ref[0] = jnp.swapaxes(ot, 0, 1)


def kernel(x, weights, channel_idx):
    w2 = weights.reshape(_F, _C)
    idx2 = channel_idx.reshape(1, _C)
    out = pl.pallas_call(
        _gemm_kernel,
        grid=(_B,),
        in_specs=[
            pl.BlockSpec((1, _C), lambda b: (0, 0)),
            pl.BlockSpec((_F, _C), lambda b: (0, 0)),
            pl.BlockSpec((1, _C, _H, _W), lambda b: (b, 0, 0, 0)),
        ],
        out_specs=pl.BlockSpec((1, _F, _H, _W), lambda b: (b, 0, 0, 0)),
        out_shape=jax.ShapeDtypeStruct((_B, _F, _H, _W), jnp.float32),
    )(idx2, w2, x)
    return out
